# baseline (speedup 1.0000x reference)
import functools

import jax
import jax.numpy as jnp
from jax import lax
from jax.experimental import pallas as pl
from jax.experimental.pallas import tpu as pltpu
from jax.experimental.pallas import tpu_sc as plsc

B, S = 4, 2048
D = 128
EPS = 1e-07

NC, NS = 2, 16
NW = NC * NS
NT = B * S
TPW = NT // NW
CHUNK = 128
NCH = TPW // CHUNK


def _gather_body(ids_hbm, wt_hbm, out_hbm, idx_v, rows_v, sem_g, sem_o):
    c = lax.axis_index("c")
    s = lax.axis_index("s")
    wid = s * NC + c
    base = wid * TPW
    b = wid // (S // TPW)
    pbase = lax.rem(wid, S // TPW) * TPW

    pltpu.sync_copy(ids_hbm.at[b, pl.ds(pbase, TPW)], idx_v)
    gathers = [
        pltpu.async_copy(wt_hbm.at[idx_v.at[pl.ds(j * CHUNK, CHUNK)]],
                         rows_v.at[pl.ds(j * CHUNK, CHUNK)], sem_g.at[j])
        for j in range(NCH)
    ]
    outs = []
    for j in range(NCH):
        gathers[j].wait()
        outs.append(
            pltpu.async_copy(rows_v.at[pl.ds(j * CHUNK, CHUNK)],
                             out_hbm.at[b, pl.ds(pbase + j * CHUNK, CHUNK)],
                             sem_o))
    for cp in outs:
        cp.wait()


def _sc_gather(ids, word_table):
    run = functools.partial(
        pl.kernel,
        out_type=jax.ShapeDtypeStruct((B, S, D), jnp.float32),
        mesh=plsc.VectorSubcoreMesh(core_axis_name="c", subcore_axis_name="s"),
        scratch_types=[
            pltpu.VMEM((TPW,), jnp.int32),
            pltpu.VMEM((TPW, D), jnp.float32),
            pltpu.SemaphoreType.DMA((NCH,)),
            pltpu.SemaphoreType.DMA,
        ],
    )(_gather_body)
    return run(ids, word_table)


BPB = 4


def _ln_body(rows_ref, pos_ref, tt_ref, g_ref, b_ref, o_ref):
    x = rows_ref[...] + pos_ref[...] + tt_ref[0:1, 0:1, :]
    mean = jnp.mean(x, axis=-1, keepdims=True)
    xc = x - mean
    var = jnp.mean(xc * xc, axis=-1, keepdims=True)
    o_ref[...] = xc * lax.rsqrt(var + EPS) * g_ref[0:1, 0:1, :] + b_ref[0:1, 0:1, :]


def _tc_layernorm(rows, pos_table, tt_table, gamma, beta):
    return pl.pallas_call(
        _ln_body,
        grid=(B // BPB,),
        in_specs=[
            pl.BlockSpec((BPB, S, D), lambda i: (i, 0, 0)),
            pl.BlockSpec((1, S, D), lambda i: (0, 0, 0)),
            pl.BlockSpec((1, 2, D), lambda i: (0, 0, 0)),
            pl.BlockSpec((1, 1, D), lambda i: (0, 0, 0)),
            pl.BlockSpec((1, 1, D), lambda i: (0, 0, 0)),
        ],
        out_specs=pl.BlockSpec((BPB, S, D), lambda i: (i, 0, 0)),
        out_shape=jax.ShapeDtypeStruct((B, S, D), jnp.float32),
    )(rows, pos_table.reshape(1, S, D),
      tt_table.reshape(1, 2, D), gamma, beta)


@jax.jit
def kernel(input_ids, word_table, pos_table, tt_table, gamma, beta):
    ids = input_ids.astype(jnp.int32)
    rows = _sc_gather(ids, word_table)
    return _tc_layernorm(rows, pos_table, tt_table,
                         gamma.reshape(1, 1, D), beta.reshape(1, 1, D))


# speedup vs baseline: 1.0361x; 1.0361x over previous
import functools

import jax
import jax.numpy as jnp
from jax import lax
from jax.experimental import pallas as pl
from jax.experimental.pallas import tpu as pltpu
from jax.experimental.pallas import tpu_sc as plsc

B, S = 4, 2048
D = 128
EPS = 1e-07

NC, NS = 2, 16
NW = NC * NS
NT = B * S
TPW = NT // NW
CHUNK = 128
NCH = TPW // CHUNK


def _gather_body(ids_hbm, wt_hbm, out_hbm, idx_v, rows_v, sem_g, sem_o):
    c = lax.axis_index("c")
    s = lax.axis_index("s")
    wid = s * NC + c
    base = wid * TPW
    b = wid // (S // TPW)
    pbase = lax.rem(wid, S // TPW) * TPW

    pltpu.sync_copy(ids_hbm.at[b, pl.ds(pbase, TPW)], idx_v)
    gathers = [
        pltpu.async_copy(wt_hbm.at[idx_v.at[pl.ds(j * CHUNK, CHUNK)]],
                         rows_v.at[pl.ds(j * CHUNK, CHUNK)], sem_g.at[j])
        for j in range(NCH)
    ]
    outs = []
    for j in range(NCH):
        gathers[j].wait()
        outs.append(
            pltpu.async_copy(rows_v.at[pl.ds(j * CHUNK, CHUNK)],
                             out_hbm.at[b, pl.ds(pbase + j * CHUNK, CHUNK)],
                             sem_o))
    for cp in outs:
        cp.wait()


def _sc_gather(ids, word_table):
    run = functools.partial(
        pl.kernel,
        out_type=jax.ShapeDtypeStruct((B, S, D), jnp.float32),
        mesh=plsc.VectorSubcoreMesh(core_axis_name="c", subcore_axis_name="s"),
        scratch_types=[
            pltpu.VMEM((TPW,), jnp.int32),
            pltpu.VMEM((TPW, D), jnp.float32),
            pltpu.SemaphoreType.DMA((NCH,)),
            pltpu.SemaphoreType.DMA,
        ],
    )(_gather_body)
    return run(ids, word_table)


BPB = 2


def _ln_body(rows_ref, pos_ref, tt_ref, g_ref, b_ref, o_ref):
    x = rows_ref[...] + pos_ref[...] + tt_ref[0:1, 0:1, :]
    mean = jnp.mean(x, axis=-1, keepdims=True)
    xc = x - mean
    var = jnp.mean(xc * xc, axis=-1, keepdims=True)
    o_ref[...] = xc * lax.rsqrt(var + EPS) * g_ref[0:1, 0:1, :] + b_ref[0:1, 0:1, :]


def _tc_layernorm(rows, pos_table, tt_table, gamma, beta):
    return pl.pallas_call(
        _ln_body,
        grid=(B // BPB,),
        in_specs=[
            pl.BlockSpec((BPB, S, D), lambda i: (i, 0, 0)),
            pl.BlockSpec((1, S, D), lambda i: (0, 0, 0)),
            pl.BlockSpec((1, 2, D), lambda i: (0, 0, 0)),
            pl.BlockSpec((1, 1, D), lambda i: (0, 0, 0)),
            pl.BlockSpec((1, 1, D), lambda i: (0, 0, 0)),
        ],
        out_specs=pl.BlockSpec((BPB, S, D), lambda i: (i, 0, 0)),
        out_shape=jax.ShapeDtypeStruct((B, S, D), jnp.float32),
    )(rows, pos_table.reshape(1, S, D),
      tt_table.reshape(1, 2, D), gamma, beta)


@jax.jit
def kernel(input_ids, word_table, pos_table, tt_table, gamma, beta):
    ids = input_ids.astype(jnp.int32)
    rows = _sc_gather(ids, word_table)
    return _tc_layernorm(rows, pos_table, tt_table,
                         gamma.reshape(1, 1, D), beta.reshape(1, 1, D))
